# Initial kernel scaffold; baseline (speedup 1.0000x reference)
import jax
import jax.numpy as jnp
from jax.experimental import pallas as pl


def kernel(confidences, wealth):
    raise NotImplementedError("write your pallas kernel here")


# speedup vs baseline: 2.2096x; 2.2096x over previous
import jax
import jax.numpy as jnp
from jax.experimental import pallas as pl

NUM_EXPERTS = 64
TOP_K = 8
_BLOCK = 2048
_TOKENS = 4 * 8192


def _body(conf_ref, w_ref, idx_ref, rw_ref, pay_ref):
    bids = conf_ref[...] * w_ref[...]
    b = bids.shape[0]
    iota = jax.lax.broadcasted_iota(jnp.int32, bids.shape, 1)
    work = bids
    idxs, tops = [], []
    for _ in range(TOP_K):
        m = jnp.max(work, axis=1, keepdims=True)
        sel = jnp.min(jnp.where(work == m, iota, NUM_EXPERTS), axis=1, keepdims=True)
        idxs.append(sel)
        tops.append(m)
        work = jnp.where(iota == sel, -jnp.inf, work)
    pay = jnp.max(work, axis=1, keepdims=True)

    mx = tops[0]
    z = jnp.sum(jnp.exp(bids - mx), axis=1, keepdims=True)
    e = [jnp.exp(t - mx) for t in tops]
    s8 = e[0]
    for ei in e[1:]:
        s8 = s8 + ei
    denom = s8 + 1e-8 * z

    idx_ref[...] = jnp.concatenate(idxs, axis=1)
    rw_ref[...] = jnp.concatenate([ei / denom for ei in e], axis=1)
    pay_ref[...] = jnp.broadcast_to(pay, (b, TOP_K))


def kernel(confidences, wealth):
    conf2d = confidences.reshape(_TOKENS, NUM_EXPERTS)
    w2d = wealth.reshape(1, NUM_EXPERTS)
    grid = _TOKENS // _BLOCK
    out_shapes = (
        jax.ShapeDtypeStruct((_TOKENS, TOP_K), jnp.int32),
        jax.ShapeDtypeStruct((_TOKENS, TOP_K), jnp.float32),
        jax.ShapeDtypeStruct((_TOKENS, TOP_K), jnp.float32),
    )
    idx, rw, pay = pl.pallas_call(
        _body,
        grid=(grid,),
        in_specs=[
            pl.BlockSpec((_BLOCK, NUM_EXPERTS), lambda i: (i, 0)),
            pl.BlockSpec((1, NUM_EXPERTS), lambda i: (0, 0)),
        ],
        out_specs=[
            pl.BlockSpec((_BLOCK, TOP_K), lambda i: (i, 0)),
            pl.BlockSpec((_BLOCK, TOP_K), lambda i: (i, 0)),
            pl.BlockSpec((_BLOCK, TOP_K), lambda i: (i, 0)),
        ],
        out_shape=out_shapes,
    )(conf2d, w2d)
    shape3 = confidences.shape[:2] + (TOP_K,)
    return idx.reshape(shape3), rw.reshape(shape3), pay.reshape(shape3)


# speedup vs baseline: 2.2400x; 1.0138x over previous
import functools

import jax
import jax.numpy as jnp
from jax import lax
from jax.experimental import pallas as pl
from jax.experimental.pallas import tpu as pltpu
from jax.experimental.pallas import tpu_sc as plsc

NUM_EXPERTS = 64
TOP_K = 8
_TOKENS = 4 * 8192
_L = 16


def _merge16(ak, ai, bk, bi):
    brk = jnp.flip(bk)
    bri = jnp.flip(bi)
    take_a = ak >= brk
    mk = jnp.maximum(ak, brk)
    mi = jnp.where(take_a, ai, bri)
    return plsc.sort_key_val(mk, mi, descending=True)


def _make_sc_call():
    info = plsc.get_sparse_core_info()
    nw = info.num_cores * info.num_subcores
    tpw = _TOKENS // nw
    mesh = plsc.VectorSubcoreMesh(core_axis_name="c", subcore_axis_name="s")

    @functools.partial(
        pl.kernel,
        mesh=mesh,
        compiler_params=pltpu.CompilerParams(needs_layout_passes=False),
        out_type=(
            jax.ShapeDtypeStruct((_TOKENS * TOP_K,), jnp.int32),
            jax.ShapeDtypeStruct((_TOKENS * TOP_K,), jnp.float32),
            jax.ShapeDtypeStruct((_TOKENS * TOP_K,), jnp.float32),
        ),
        scratch_types=[
            pltpu.VMEM((tpw // 2, NUM_EXPERTS), jnp.float32),
            pltpu.VMEM((NUM_EXPERTS,), jnp.float32),
            pltpu.VMEM((tpw * TOP_K + _L,), jnp.int32),
            pltpu.VMEM((tpw * TOP_K + _L,), jnp.float32),
            pltpu.VMEM((tpw * TOP_K + _L,), jnp.float32),
        ],
    )
    def sc_kernel(conf_hbm, w_hbm, idx_hbm, rw_hbm, pay_hbm,
                  conf_v, w_v, idx_v, rw_v, pay_v):
        wid = lax.axis_index("s") * info.num_cores + lax.axis_index("c")
        base = wid * tpw
        half = tpw // 2
        pltpu.sync_copy(w_hbm, w_v)

        lanes = lax.iota(jnp.int32, _L)
        w_regs = [w_v[pl.ds(j * _L, _L)] for j in range(4)]
        idx_regs = [lanes + j * _L for j in range(4)]
        lo_mask = lanes < TOP_K

        def chunk(c, _):
            pltpu.sync_copy(conf_hbm.at[pl.ds(base + c * half, half), :], conf_v)
            lax.fori_loop(0, half, functools.partial(body, c), 0)
            return _

        def body(c, ti, _):
            t = c * half + ti
            bids = [conf_v[ti, pl.ds(j * _L, _L)] * w_regs[j] for j in range(4)]
            srt = [plsc.sort_key_val(bids[j], idx_regs[j], descending=True)
                   for j in range(4)]
            t0k, t0i = _merge16(srt[0][0], srt[0][1], srt[1][0], srt[1][1])
            t1k, t1i = _merge16(srt[2][0], srt[2][1], srt[3][0], srt[3][1])
            topk, topi = _merge16(t0k, t0i, t1k, t1i)

            mx = jnp.max(topk)
            z = jnp.sum(jnp.exp(bids[0] - mx) + jnp.exp(bids[1] - mx)
                        + jnp.exp(bids[2] - mx) + jnp.exp(bids[3] - mx))
            e_top = jnp.exp(topk - mx)
            s8 = jnp.sum(jnp.where(lo_mask, e_top, 0.0))
            rw = e_top / (s8 + 1e-8 * z)
            pay = jnp.sum(jnp.where(lanes == TOP_K, topk, 0.0))
            pay_vec = jnp.full((_L,), 1.0, jnp.float32) * pay

            off = t * TOP_K
            plsc.store_compressed(idx_v.at[pl.ds(off, _L)], topi, mask=lo_mask)
            plsc.store_compressed(rw_v.at[pl.ds(off, _L)], rw, mask=lo_mask)
            plsc.store_compressed(pay_v.at[pl.ds(off, _L)], pay_vec, mask=lo_mask)
            return _

        lax.fori_loop(0, 2, chunk, 0)

        obase = base * TOP_K
        n = tpw * TOP_K
        pltpu.sync_copy(idx_v.at[pl.ds(0, n)], idx_hbm.at[pl.ds(obase, n)])
        pltpu.sync_copy(rw_v.at[pl.ds(0, n)], rw_hbm.at[pl.ds(obase, n)])
        pltpu.sync_copy(pay_v.at[pl.ds(0, n)], pay_hbm.at[pl.ds(obase, n)])

    return sc_kernel


_sc_call = _make_sc_call()


def kernel(confidences, wealth):
    conf2d = confidences.reshape(_TOKENS, NUM_EXPERTS)
    idx, rw, pay = _sc_call(conf2d, wealth)
    shape3 = confidences.shape[:2] + (TOP_K,)
    return idx.reshape(shape3), rw.reshape(shape3), pay.reshape(shape3)


# speedup vs baseline: 3.4314x; 1.5318x over previous
import functools

import jax
import jax.numpy as jnp
from jax import lax
from jax.experimental import pallas as pl
from jax.experimental.pallas import tpu as pltpu
from jax.experimental.pallas import tpu_sc as plsc

NUM_EXPERTS = 64
TOP_K = 8
_TOKENS = 4 * 8192
_L = 16


def _merge16(ak, ai, bk, bi):
    brk = jnp.flip(bk)
    bri = jnp.flip(bi)
    take_a = ak >= brk
    mk = jnp.maximum(ak, brk)
    mi = jnp.where(take_a, ai, bri)
    return plsc.sort_key_val(mk, mi, descending=True)


def _make_sc_call():
    info = plsc.get_sparse_core_info()
    nw = info.num_cores * info.num_subcores
    tpw = _TOKENS // nw
    mesh = plsc.VectorSubcoreMesh(core_axis_name="c", subcore_axis_name="s")

    @functools.partial(
        pl.kernel,
        mesh=mesh,
        compiler_params=pltpu.CompilerParams(needs_layout_passes=False),
        out_type=(
            jax.ShapeDtypeStruct((_TOKENS * TOP_K,), jnp.int32),
            jax.ShapeDtypeStruct((_TOKENS * TOP_K,), jnp.float32),
            jax.ShapeDtypeStruct((_TOKENS * TOP_K,), jnp.float32),
        ),
        scratch_types=[
            pltpu.VMEM((tpw // 2, NUM_EXPERTS), jnp.float32),
            pltpu.VMEM((NUM_EXPERTS,), jnp.float32),
            pltpu.VMEM((tpw * TOP_K + _L,), jnp.int32),
            pltpu.VMEM((tpw * TOP_K + _L,), jnp.float32),
            pltpu.VMEM((tpw * TOP_K + _L,), jnp.float32),
        ],
    )
    def sc_kernel(conf_hbm, w_hbm, idx_hbm, rw_hbm, pay_hbm,
                  conf_v, w_v, idx_v, rw_v, pay_v):
        wid = lax.axis_index("s") * info.num_cores + lax.axis_index("c")
        base = wid * tpw
        half = tpw // 2
        pltpu.sync_copy(w_hbm, w_v)

        lanes = lax.iota(jnp.int32, _L)
        w_regs = [w_v[pl.ds(j * _L, _L)] for j in range(4)]
        idx_regs = [lanes + j * _L for j in range(4)]
        lo_mask = lanes < TOP_K

        def chunk(c, _):
            pltpu.sync_copy(conf_hbm.at[pl.ds(base + c * half, half), :], conf_v)
            plsc.parallel_loop(0, half, 1, unroll=4)(functools.partial(body, c))
            return _

        def body(c, ti):
            t = c * half + ti
            bids = [conf_v[ti, pl.ds(j * _L, _L)] * w_regs[j] for j in range(4)]
            srt = [plsc.sort_key_val(bids[j], idx_regs[j], descending=True)
                   for j in range(4)]
            t0k, t0i = _merge16(srt[0][0], srt[0][1], srt[1][0], srt[1][1])
            t1k, t1i = _merge16(srt[2][0], srt[2][1], srt[3][0], srt[3][1])
            topk, topi = _merge16(t0k, t0i, t1k, t1i)

            mx = jnp.max(topk)
            z = jnp.sum(jnp.exp(bids[0] - mx) + jnp.exp(bids[1] - mx)
                        + jnp.exp(bids[2] - mx) + jnp.exp(bids[3] - mx))
            e_top = jnp.exp(topk - mx)
            s8 = jnp.sum(jnp.where(lo_mask, e_top, 0.0))
            rw = e_top / (s8 + 1e-8 * z)
            pay = jnp.sum(jnp.where(lanes == TOP_K, topk, 0.0))
            pay_vec = jnp.full((_L,), 1.0, jnp.float32) * pay

            off = t * TOP_K
            plsc.store_compressed(idx_v.at[pl.ds(off, _L)], topi, mask=lo_mask)
            plsc.store_compressed(rw_v.at[pl.ds(off, _L)], rw, mask=lo_mask)
            plsc.store_compressed(pay_v.at[pl.ds(off, _L)], pay_vec, mask=lo_mask)

        lax.fori_loop(0, 2, chunk, 0)

        obase = base * TOP_K
        n = tpw * TOP_K
        pltpu.sync_copy(idx_v.at[pl.ds(0, n)], idx_hbm.at[pl.ds(obase, n)])
        pltpu.sync_copy(rw_v.at[pl.ds(0, n)], rw_hbm.at[pl.ds(obase, n)])
        pltpu.sync_copy(pay_v.at[pl.ds(0, n)], pay_hbm.at[pl.ds(obase, n)])

    return sc_kernel


_sc_call = _make_sc_call()


def kernel(confidences, wealth):
    conf2d = confidences.reshape(_TOKENS, NUM_EXPERTS)
    idx, rw, pay = _sc_call(conf2d, wealth)
    shape3 = confidences.shape[:2] + (TOP_K,)
    return idx.reshape(shape3), rw.reshape(shape3), pay.reshape(shape3)


# speedup vs baseline: 3.9532x; 1.1521x over previous
import functools

import jax
import jax.numpy as jnp
from jax import lax
from jax.experimental import pallas as pl
from jax.experimental.pallas import tpu as pltpu
from jax.experimental.pallas import tpu_sc as plsc

NUM_EXPERTS = 64
TOP_K = 8
_B = 4
_S = 8192
_TOKENS = _B * _S
_L = 16
_HALF = 128


def _merge16(ak, ai, bk, bi):
    brk = jnp.flip(bk)
    bri = jnp.flip(bi)
    take_a = ak >= brk
    mk = jnp.maximum(ak, brk)
    mi = jnp.where(take_a, ai, bri)
    return plsc.sort_key_val(mk, mi, descending=True)


def _make_sc_call():
    info = plsc.get_sparse_core_info()
    nw = info.num_cores * info.num_subcores
    tpw = _TOKENS // nw
    nchunks = tpw // _HALF
    mesh = plsc.VectorSubcoreMesh(core_axis_name="c", subcore_axis_name="s")

    @functools.partial(
        pl.kernel,
        mesh=mesh,
        compiler_params=pltpu.CompilerParams(needs_layout_passes=False),
        out_type=(
            jax.ShapeDtypeStruct((_B, _S, TOP_K), jnp.int32),
            jax.ShapeDtypeStruct((_B, _S, TOP_K), jnp.float32),
            jax.ShapeDtypeStruct((_B, _S, TOP_K), jnp.float32),
        ),
        scratch_types=[
            pltpu.VMEM((_HALF, NUM_EXPERTS), jnp.float32),
            pltpu.VMEM((NUM_EXPERTS,), jnp.float32),
            pltpu.VMEM((_HALF, TOP_K), jnp.int32),
            pltpu.VMEM((_HALF, TOP_K), jnp.float32),
            pltpu.VMEM((_HALF, TOP_K), jnp.float32),
        ],
    )
    def sc_kernel(conf_hbm, w_hbm, idx_hbm, rw_hbm, pay_hbm,
                  conf_v, w_v, idx_v, rw_v, pay_v):
        wid = lax.axis_index("s") * info.num_cores + lax.axis_index("c")
        base = wid * tpw
        pltpu.sync_copy(w_hbm, w_v)

        lanes = lax.iota(jnp.int32, _L)
        w_regs = [w_v[pl.ds(j * _L, _L)] for j in range(4)]
        idx_regs = [lanes + j * _L for j in range(4)]
        lo_mask = lanes < TOP_K

        def body(ti):
            bids = [conf_v[ti, pl.ds(j * _L, _L)] * w_regs[j] for j in range(4)]
            srt = [plsc.sort_key_val(bids[j], idx_regs[j], descending=True)
                   for j in range(4)]
            t0k, t0i = _merge16(srt[0][0], srt[0][1], srt[1][0], srt[1][1])
            t1k, t1i = _merge16(srt[2][0], srt[2][1], srt[3][0], srt[3][1])
            topk, topi = _merge16(t0k, t0i, t1k, t1i)

            mx = jnp.max(topk)
            z = jnp.sum(jnp.exp(bids[0] - mx) + jnp.exp(bids[1] - mx)
                        + jnp.exp(bids[2] - mx) + jnp.exp(bids[3] - mx))
            e_top = jnp.exp(topk - mx)
            s8 = jnp.sum(jnp.where(lo_mask, e_top, 0.0))
            rw = e_top / (s8 + 1e-8 * z)
            pay = jnp.sum(jnp.where(lanes == TOP_K, topk, 0.0))
            pay_vec = lanes * 0.0 + pay

            rows = lanes * 0 + ti
            plsc.store_scatter(idx_v, [rows, lanes], topi, mask=lo_mask)
            plsc.store_scatter(rw_v, [rows, lanes], rw, mask=lo_mask)
            plsc.store_scatter(pay_v, [rows, lanes], pay_vec, mask=lo_mask)

        def chunk(c, _):
            tok0 = base + c * _HALF
            b_idx = tok0 // _S
            s0 = tok0 - b_idx * _S
            pltpu.sync_copy(conf_hbm.at[pl.ds(tok0, _HALF), :], conf_v)
            plsc.parallel_loop(0, _HALF, 1, unroll=4)(body)
            pltpu.sync_copy(idx_v, idx_hbm.at[b_idx, pl.ds(s0, _HALF), :])
            pltpu.sync_copy(rw_v, rw_hbm.at[b_idx, pl.ds(s0, _HALF), :])
            pltpu.sync_copy(pay_v, pay_hbm.at[b_idx, pl.ds(s0, _HALF), :])
            return _

        lax.fori_loop(0, nchunks, chunk, 0)

    return sc_kernel


_sc_call = _make_sc_call()


def kernel(confidences, wealth):
    conf2d = confidences.reshape(_TOKENS, NUM_EXPERTS)
    return _sc_call(conf2d, wealth)


# speedup vs baseline: 4.2762x; 1.0817x over previous
import functools

import jax
import jax.numpy as jnp
from jax import lax
from jax.experimental import pallas as pl
from jax.experimental.pallas import tpu as pltpu
from jax.experimental.pallas import tpu_sc as plsc

NUM_EXPERTS = 64
TOP_K = 8
_B = 4
_S = 8192
_TOKENS = _B * _S
_L = 16
_CH = 64


def _merge16(ak, ai, bk, bi):
    brk = jnp.flip(bk)
    bri = jnp.flip(bi)
    take_a = ak >= brk
    mk = jnp.maximum(ak, brk)
    mi = jnp.where(take_a, ai, bri)
    return plsc.sort_key_val(mk, mi, descending=True)


def _make_sc_call():
    info = plsc.get_sparse_core_info()
    nw = info.num_cores * info.num_subcores
    tpw = _TOKENS // nw
    nchunks = tpw // _CH
    mesh = plsc.VectorSubcoreMesh(core_axis_name="c", subcore_axis_name="s")

    @functools.partial(
        pl.kernel,
        mesh=mesh,
        compiler_params=pltpu.CompilerParams(needs_layout_passes=False),
        out_type=(
            jax.ShapeDtypeStruct((_B, _S, TOP_K), jnp.int32),
            jax.ShapeDtypeStruct((_B, _S, TOP_K), jnp.float32),
            jax.ShapeDtypeStruct((_B, _S, TOP_K), jnp.float32),
        ),
        scratch_types=[
            pltpu.VMEM((2, _CH, NUM_EXPERTS), jnp.float32),
            pltpu.VMEM((NUM_EXPERTS,), jnp.float32),
            pltpu.VMEM((2, _CH, TOP_K), jnp.int32),
            pltpu.VMEM((2, _CH, TOP_K), jnp.float32),
            pltpu.VMEM((2, _CH, TOP_K), jnp.float32),
            pltpu.SemaphoreType.DMA,
            pltpu.SemaphoreType.DMA,
            pltpu.SemaphoreType.DMA,
            pltpu.SemaphoreType.DMA,
        ],
    )
    def sc_kernel(conf_hbm, w_hbm, idx_hbm, rw_hbm, pay_hbm,
                  conf_v, w_v, idx_v, rw_v, pay_v,
                  in_sem, oi_sem, or_sem, op_sem):
        wid = lax.axis_index("s") * info.num_cores + lax.axis_index("c")
        base = wid * tpw
        pltpu.sync_copy(w_hbm, w_v)

        lanes = lax.iota(jnp.int32, _L)
        w_regs = [w_v[pl.ds(j * _L, _L)] for j in range(4)]
        idx_regs = [lanes + j * _L for j in range(4)]
        lo_mask = lanes < TOP_K

        def in_src(c):
            return conf_hbm.at[pl.ds(base + c * _CH, _CH), :]

        def out_dst(hbm, c):
            tok0 = base + c * _CH
            b_idx = tok0 // _S
            return hbm.at[b_idx, pl.ds(tok0 - b_idx * _S, _CH), :]

        def body(b, ti):
            bids = [conf_v[b, ti, pl.ds(j * _L, _L)] * w_regs[j]
                    for j in range(4)]
            srt = [plsc.sort_key_val(bids[j], idx_regs[j], descending=True)
                   for j in range(4)]
            t0k, t0i = _merge16(srt[0][0], srt[0][1], srt[1][0], srt[1][1])
            t1k, t1i = _merge16(srt[2][0], srt[2][1], srt[3][0], srt[3][1])
            topk, topi = _merge16(t0k, t0i, t1k, t1i)

            z = jnp.sum(jnp.exp(bids[0]) + jnp.exp(bids[1])
                        + jnp.exp(bids[2]) + jnp.exp(bids[3]))
            e_top = jnp.exp(topk)
            s8 = jnp.sum(jnp.where(lo_mask, e_top, 0.0))
            rw = e_top / (s8 + 1e-8 * z)
            pay = jnp.sum(jnp.where(lanes == TOP_K, topk, 0.0))
            pay_vec = lanes * 0.0 + pay

            rows = lanes * 0 + ti
            plsc.store_scatter(idx_v.at[b], [rows, lanes], topi, mask=lo_mask)
            plsc.store_scatter(rw_v.at[b], [rows, lanes], rw, mask=lo_mask)
            plsc.store_scatter(pay_v.at[b], [rows, lanes], pay_vec,
                               mask=lo_mask)

        pltpu.async_copy(in_src(0), conf_v.at[0], in_sem)
        for c in range(nchunks):
            b = c & 1
            pltpu.make_async_copy(in_src(c), conf_v.at[b], in_sem).wait()
            if c + 1 < nchunks:
                pltpu.async_copy(in_src(c + 1), conf_v.at[1 - b], in_sem)
            if c >= 2:
                pltpu.make_async_copy(idx_v.at[b], out_dst(idx_hbm, c - 2),
                                      oi_sem).wait()
                pltpu.make_async_copy(rw_v.at[b], out_dst(rw_hbm, c - 2),
                                      or_sem).wait()
                pltpu.make_async_copy(pay_v.at[b], out_dst(pay_hbm, c - 2),
                                      op_sem).wait()
            plsc.parallel_loop(0, _CH, 1, unroll=4)(
                functools.partial(body, b))
            pltpu.async_copy(idx_v.at[b], out_dst(idx_hbm, c), oi_sem)
            pltpu.async_copy(rw_v.at[b], out_dst(rw_hbm, c), or_sem)
            pltpu.async_copy(pay_v.at[b], out_dst(pay_hbm, c), op_sem)
        for c in (nchunks - 2, nchunks - 1):
            b = c & 1
            pltpu.make_async_copy(idx_v.at[b], out_dst(idx_hbm, c),
                                  oi_sem).wait()
            pltpu.make_async_copy(rw_v.at[b], out_dst(rw_hbm, c),
                                  or_sem).wait()
            pltpu.make_async_copy(pay_v.at[b], out_dst(pay_hbm, c),
                                  op_sem).wait()

    return sc_kernel


_sc_call = _make_sc_call()


def kernel(confidences, wealth):
    conf2d = confidences.reshape(_TOKENS, NUM_EXPERTS)
    return _sc_call(conf2d, wealth)
